# baseline (device time: 34757 ns/iter reference)
import jax
import jax.numpy as jnp
from jax import lax
from jax.experimental import pallas as pl
from jax.experimental.pallas import tpu as pltpu

N_BLOCKS = 8


def kernel(x):
    m, n = x.shape
    mb = m // N_BLOCKS

    def body(x_ref, out_ref, comm_ref, send_sem, recv_sem):
        i = pl.program_id(0)
        my_x = lax.axis_index("x")
        my_y = lax.axis_index("y")
        nbr = (my_x, 1 - my_y)

        @pl.when(i == 0)
        def _():
            barrier_sem = pltpu.get_barrier_semaphore()
            pl.semaphore_signal(
                barrier_sem, inc=1, device_id=nbr,
                device_id_type=pl.DeviceIdType.MESH,
            )
            pl.semaphore_wait(barrier_sem, 1)

        comm_ref[0, pl.ds(i * mb, mb), :] = jnp.sum(
            x_ref[...], axis=1, keepdims=True
        )

        @pl.when(i == N_BLOCKS - 1)
        def _():
            rdma = pltpu.make_async_remote_copy(
                src_ref=comm_ref.at[0],
                dst_ref=comm_ref.at[1],
                send_sem=send_sem,
                recv_sem=recv_sem,
                device_id=nbr,
                device_id_type=pl.DeviceIdType.MESH,
            )
            rdma.start()
            rdma.wait()
            out_ref[...] = comm_ref[0] + comm_ref[1]

    return pl.pallas_call(
        body,
        grid=(N_BLOCKS,),
        out_shape=jax.ShapeDtypeStruct((m, 1), jnp.float32),
        in_specs=[
            pl.BlockSpec((mb, n), lambda i: (i, 0), memory_space=pltpu.VMEM)
        ],
        out_specs=pl.BlockSpec((m, 1), lambda i: (0, 0), memory_space=pltpu.VMEM),
        scratch_shapes=[
            pltpu.VMEM((2, m, 1), jnp.float32),
            pltpu.SemaphoreType.DMA,
            pltpu.SemaphoreType.DMA,
        ],
        compiler_params=pltpu.CompilerParams(collective_id=0),
    )(x)


# device time: 10274 ns/iter; 3.3830x vs baseline; 3.3830x over previous
import jax
import jax.numpy as jnp
from jax import lax
from jax.experimental import pallas as pl
from jax.experimental.pallas import tpu as pltpu

N_BLOCKS = 8


def kernel(x):
    x = pltpu.with_memory_space_constraint(x, pltpu.MemorySpace.HBM)
    m, n = x.shape
    mb = m // N_BLOCKS
    pr = mb // 128
    rows = m // 128
    half = N_BLOCKS // 2
    hrows = rows // 2

    def body(x_ref, out_ref, vbuf, comm_ref, dsems, send_sems, recv_sems):
        my_x = lax.axis_index("x")
        my_y = lax.axis_index("y")
        nbr = (my_x, 1 - my_y)

        def blk_cp(i):
            return pltpu.make_async_copy(
                x_ref.at[pl.ds(i * mb, mb), :], vbuf.at[i], dsems.at[i])

        for i in range(N_BLOCKS):
            blk_cp(i).start()

        bsem = pltpu.get_barrier_semaphore()
        pl.semaphore_signal(bsem, inc=1, device_id=nbr,
                            device_id_type=pl.DeviceIdType.MESH)
        pl.semaphore_wait(bsem, 1)

        def exch(slot, row0, nrows):
            rdma = pltpu.make_async_remote_copy(
                src_ref=comm_ref.at[0, pl.ds(row0, nrows), :],
                dst_ref=comm_ref.at[1, pl.ds(row0, nrows), :],
                send_sem=send_sems.at[slot],
                recv_sem=recv_sems.at[slot],
                device_id=nbr, device_id_type=pl.DeviceIdType.MESH)
            rdma.start()
            return rdma

        rdmas = []
        for i in range(N_BLOCKS):
            blk_cp(i).wait()
            p = jnp.sum(vbuf[i], axis=1).reshape(pr, 128)
            comm_ref[0, pl.ds(i * pr, pr), :] = p
            if i == half - 1:
                rdmas.append(exch(0, 0, hrows))
        rdmas.append(exch(1, hrows, hrows))

        for r in rdmas:
            r.wait()
        out_ref[...] = comm_ref[0] + comm_ref[1]

    packed = pl.pallas_call(
        body,
        out_shape=jax.ShapeDtypeStruct((rows, 128), jnp.float32),
        in_specs=[pl.BlockSpec(memory_space=pl.ANY)],
        out_specs=pl.BlockSpec(memory_space=pltpu.VMEM),
        scratch_shapes=[
            pltpu.VMEM((N_BLOCKS, mb, n), jnp.float32),
            pltpu.VMEM((2, rows, 128), jnp.float32),
            pltpu.SemaphoreType.DMA((N_BLOCKS,)),
            pltpu.SemaphoreType.DMA((2,)),
            pltpu.SemaphoreType.DMA((2,)),
        ],
        compiler_params=pltpu.CompilerParams(collective_id=0),
    )(x)
    return packed.reshape(m, 1)
